# baseline (speedup 1.0000x reference)
import jax
import jax.numpy as jnp
from jax.experimental import pallas as pl

N_NODES = 325
NP = 384
IN_DIM = 2
UNITS = 64
BATCH = 1024
BT = 32
GRID = BATCH // BT


def _nmix(x, STp):
    c, b, n = x.shape
    y = jax.lax.dot_general(x.reshape(c * b, n), STp,
                            (((1,), (0,)), ((), ())),
                            preferred_element_type=jnp.float32)
    return y.reshape(c, b, n)


def _cmix(v, x):
    c, b, n = x.shape
    y = jax.lax.dot_general(v, x.reshape(c, b * n),
                            (((0,), (0,)), ((), ())),
                            preferred_element_type=jnp.float32)
    return y.reshape(v.shape[1], b, n)


def _dcgru_kernel(xi_ref, h_ref, ST_ref,
                  vh_ru_ref, vx_ru_ref, bru_ref,
                  vh_c_ref, vx_c_ref, bc_ref,
                  out_ref):
    bf = jnp.bfloat16
    ST = ST_ref[...]
    hb = h_ref[...]
    xib = xi_ref[...]

    xi1 = _nmix(xib, ST)
    xi2 = _nmix(xi1.astype(bf), ST)
    xis = (xib, xi1.astype(bf), xi2.astype(bf))

    def gconv(st_b, vh_ref, vx_ref, b_ref):
        s1 = _nmix(st_b, ST)
        s2 = _nmix(s1.astype(bf), ST)
        acc = (_cmix(vh_ref[0], st_b)
               + _cmix(vh_ref[1], s1.astype(bf))
               + _cmix(vh_ref[2], s2.astype(bf))
               + _cmix(vx_ref[0], xis[0])
               + _cmix(vx_ref[1], xis[1])
               + _cmix(vx_ref[2], xis[2]))
        return acc + b_ref[...]

    ru = jax.nn.sigmoid(gconv(hb, vh_ru_ref, vx_ru_ref, bru_ref))
    r = ru[:UNITS]
    u = ru[UNITS:]

    st = (r * hb).astype(bf)
    c = jnp.tanh(gconv(st, vh_c_ref, vx_c_ref, bc_ref))

    out_ref[...] = u * hb + (1.0 - u) * c


def _fold_weights(W, out_dim):
    Wm = W.reshape(IN_DIM + UNITS, 3, out_dim)
    V0 = Wm[:, 0, :] - Wm[:, 2, :]
    V1 = Wm[:, 1, :]
    V2 = 2.0 * Wm[:, 2, :]
    V = jnp.stack([V0, V1, V2])
    return V[:, IN_DIM:, :], V[:, :IN_DIM, :]


@jax.jit
def kernel(inputs, hidden_state, support, W_ru, b_ru, W_c, b_c):
    B, N, U, bf = BATCH, N_NODES, UNITS, jnp.bfloat16
    pad = ((0, 0), (0, 0), (0, NP - N))
    xiT = jnp.pad(inputs.reshape(B, N, IN_DIM).transpose(2, 0, 1),
                  pad).astype(bf)
    hT = jnp.pad(hidden_state[0].reshape(B, N, U).transpose(2, 0, 1),
                 pad).astype(bf)

    vh_ru, vx_ru = _fold_weights(W_ru, 2 * U)
    vh_c, vx_c = _fold_weights(W_c, U)
    bru = jnp.broadcast_to(b_ru[:, None, None], (2 * U, 1, NP))
    bc = jnp.broadcast_to(b_c[:, None, None], (U, 1, NP))
    ST = jnp.pad(support.T, ((0, NP - N), (0, NP - N))).astype(bf)

    full = lambda a: pl.BlockSpec(a.shape, lambda i: (0,) * a.ndim)
    bspec = lambda c: pl.BlockSpec((c, BT, NP), lambda i: (0, i, 0))

    y = pl.pallas_call(
        _dcgru_kernel,
        grid=(GRID,),
        in_specs=[
            bspec(IN_DIM), bspec(U),
            full(ST), full(vh_ru.astype(bf)), full(vx_ru.astype(bf)),
            full(bru), full(vh_c.astype(bf)), full(vx_c.astype(bf)),
            full(bc),
        ],
        out_specs=bspec(U),
        out_shape=jax.ShapeDtypeStruct((U, B, NP), jnp.float32),
    )(xiT, hT, ST, vh_ru.astype(bf), vx_ru.astype(bf), bru,
      vh_c.astype(bf), vx_c.astype(bf), bc)

    t = y[:, :, :N].transpose(1, 2, 0)
    return (t.reshape(B, N * U), t.reshape(1, B, N * U))


# speedup vs baseline: 1.6807x; 1.6807x over previous
import jax
import jax.numpy as jnp
from jax.experimental import pallas as pl

N_NODES = 325
IN_DIM = 2
UNITS = 64
BATCH = 1024
BT = 16
GRID = BATCH // BT


def _nmix(x, ST):
    b, c, n = x.shape
    y = jax.lax.dot_general(x.reshape(b * c, n), ST, (((1,), (0,)), ((), ())),
                            preferred_element_type=jnp.float32)
    return y.reshape(b, c, n)


def _cmix(w, x):
    return jax.lax.dot_general(w, x, (((1,), (1,)), ((0,), (0,))),
                               preferred_element_type=jnp.float32)


def _dcgru_kernel(xi_ref, h_ref, ST_ref,
                  vh_ru_ref, vx_ru_ref, bru_ref,
                  vh_c_ref, vx_c_ref, bc_ref,
                  out_ref):
    bf = jnp.bfloat16
    ST = ST_ref[...]
    hb = h_ref[...]
    xib = xi_ref[...]

    def bcast(ref, m):
        return jnp.broadcast_to(ref[m][None], (BT,) + ref.shape[1:])

    xi1 = _nmix(xib, ST)
    xi2 = _nmix(xi1.astype(bf), ST)
    xis = (xib, xi1.astype(bf), xi2.astype(bf))

    def gconv(st_b, vh_ref, vx_ref, b_ref):
        s1 = _nmix(st_b, ST)
        s2 = _nmix(s1.astype(bf), ST)
        acc = (_cmix(bcast(vh_ref, 0), st_b)
               + _cmix(bcast(vh_ref, 1), s1.astype(bf))
               + _cmix(bcast(vh_ref, 2), s2.astype(bf))
               + _cmix(bcast(vx_ref, 0), xis[0])
               + _cmix(bcast(vx_ref, 1), xis[1])
               + _cmix(bcast(vx_ref, 2), xis[2]))
        return acc + b_ref[...]

    ru = jax.nn.sigmoid(gconv(hb, vh_ru_ref, vx_ru_ref, bru_ref))
    r = ru[:, :UNITS, :]
    u = ru[:, UNITS:, :]

    st = (r * hb).astype(bf)
    c = jnp.tanh(gconv(st, vh_c_ref, vx_c_ref, bc_ref))

    out_ref[...] = u * hb + (1.0 - u) * c


def _fold_weights(W, out_dim):
    Wm = W.reshape(IN_DIM + UNITS, 3, out_dim)
    V0 = Wm[:, 0, :] - Wm[:, 2, :]
    V1 = Wm[:, 1, :]
    V2 = 2.0 * Wm[:, 2, :]
    V = jnp.stack([V0, V1, V2])
    return V[:, IN_DIM:, :], V[:, :IN_DIM, :]


@jax.jit
def kernel(inputs, hidden_state, support, W_ru, b_ru, W_c, b_c):
    B, N, U, bf = BATCH, N_NODES, UNITS, jnp.bfloat16
    xiT = inputs.reshape(B, N, IN_DIM).transpose(0, 2, 1).astype(bf)
    hT = hidden_state[0].reshape(B, N, U).transpose(0, 2, 1).astype(bf)

    vh_ru, vx_ru = _fold_weights(W_ru, 2 * U)
    vh_c, vx_c = _fold_weights(W_c, U)
    bru = b_ru.reshape(1, 2 * U, 1)
    bc = b_c.reshape(1, U, 1)
    ST = support.T.astype(bf)

    full = lambda a: pl.BlockSpec(a.shape, lambda i: (0,) * a.ndim)
    bspec = lambda c: pl.BlockSpec((BT, c, N), lambda i: (i, 0, 0))

    y = pl.pallas_call(
        _dcgru_kernel,
        grid=(GRID,),
        in_specs=[
            bspec(IN_DIM), bspec(U),
            full(ST), full(vh_ru.astype(bf)), full(vx_ru.astype(bf)),
            full(bru), full(vh_c.astype(bf)), full(vx_c.astype(bf)),
            full(bc),
        ],
        out_specs=bspec(U),
        out_shape=jax.ShapeDtypeStruct((B, U, N), jnp.float32),
    )(xiT, hT, ST, vh_ru.astype(bf), vx_ru.astype(bf), bru,
      vh_c.astype(bf), vx_c.astype(bf), bc)

    output = y.transpose(0, 2, 1).reshape(B, N * U)
    return (output, output[None])
